# baseline (speedup 1.0000x reference)
import functools

import jax
import jax.numpy as jnp
from jax import lax
from jax.experimental import pallas as pl
from jax.experimental.pallas import tpu as pltpu
from jax.experimental.pallas import tpu_sc as plsc

M = 8
K = 1024
D = 256
SUB = D // M

TN = 256

NW = 32
CH = 128


def _csq_body(m2cb_ref, csq_ref):
    for m in range(M):
        c2 = m2cb_ref[m]
        csq_ref[m, :] = 0.25 * jnp.sum(c2 * c2, axis=1)


def _tc_body(x_ref, m2cb_ref, csq_ref, iota_ref, codes_ref, flat_t_ref):
    xs = x_ref[...]
    cross = jnp.concatenate(
        [jax.lax.dot_general(
            xs[:, m * SUB:(m + 1) * SUB], m2cb_ref[m],
            (((1,), (1,)), ((), ())), preferred_element_type=jnp.float32)
         for m in range(M)], axis=1)
    dist = cross.reshape(TN, M, K) + csq_ref[...][None]
    dmin = jnp.min(dist, axis=2, keepdims=True)
    codesf = jnp.min(
        jnp.where(dist == dmin, iota_ref[...][None], float(K)), axis=2)
    codes = codesf.astype(jnp.int32)
    codes_ref[...] = codes
    flat = codes + jax.lax.broadcasted_iota(jnp.int32, (TN, M), 1) * K
    flat_t_ref[...] = flat.T


def _sc_gather(idx_hbm, table_hbm, xr_out, side_out, idx_v, rows_v, sem):
    tpw = rows_v.shape[0] // M
    wid = lax.axis_index("s") * 2 + lax.axis_index("c")
    t0 = wid * tpw
    pltpu.sync_copy(idx_hbm.at[wid], idx_v)
    copies = [
        pltpu.async_copy(table_hbm.at[idx_v.at[j]],
                         rows_v.at[pl.ds(j * CH, CH)], sem)
        for j in range(idx_v.shape[0])
    ]
    for c in copies:
        c.wait()
    for m in range(M):
        seg = rows_v.at[pl.ds(m * tpw, tpw), :]
        pltpu.sync_copy(seg, side_out.at[m, pl.ds(t0, tpw), :])
        pltpu.sync_copy(seg, xr_out.at[pl.ds(t0, tpw), pl.ds(m * SUB, SUB)])


@jax.jit
def kernel(x, codebook):
    n = x.shape[0]
    grid = n // TN
    iota2d = jax.lax.broadcasted_iota(jnp.float32, (M, K), 1)
    m2cb = -2.0 * codebook

    csq = pl.pallas_call(
        _csq_body,
        out_shape=jax.ShapeDtypeStruct((M, K), jnp.float32),
    )(m2cb)

    codes, flat_mn = pl.pallas_call(
        _tc_body,
        grid=(grid,),
        in_specs=[
            pl.BlockSpec((TN, D), lambda i: (i, 0)),
            pl.BlockSpec((M, K, SUB), lambda i: (0, 0, 0)),
            pl.BlockSpec((M, K), lambda i: (0, 0)),
            pl.BlockSpec((M, K), lambda i: (0, 0)),
        ],
        out_specs=[
            pl.BlockSpec((TN, M), lambda i: (i, 0)),
            pl.BlockSpec((M, TN), lambda i: (0, i)),
        ],
        out_shape=[
            jax.ShapeDtypeStruct((n, M), jnp.int32),
            jax.ShapeDtypeStruct((M, n), jnp.int32),
        ],
    )(x, m2cb, csq, iota2d)

    tpw = n // NW
    rpw = tpw * M
    nch = rpw // CH
    idx_w = flat_mn.reshape(M, NW, tpw).swapaxes(0, 1).reshape(NW, nch, CH)
    table = codebook.reshape(M * K, SUB)

    mesh = plsc.VectorSubcoreMesh(core_axis_name="c", subcore_axis_name="s")
    x_recon, side = pl.kernel(
        _sc_gather,
        out_type=[
            jax.ShapeDtypeStruct((n, D), jnp.float32),
            jax.ShapeDtypeStruct((M, n, SUB), jnp.float32),
        ],
        mesh=mesh,
        scratch_types=[
            pltpu.VMEM((nch, CH), jnp.int32),
            pltpu.VMEM((rpw, SUB), jnp.float32),
            pltpu.SemaphoreType.DMA,
        ],
        compiler_params=pltpu.CompilerParams(use_tc_tiling_on_sc=False),
    )(idx_w, table)

    return (x_recon, codes, side)


# speedup vs baseline: 1.0285x; 1.0285x over previous
import functools

import jax
import jax.numpy as jnp
from jax import lax
from jax.experimental import pallas as pl
from jax.experimental.pallas import tpu as pltpu
from jax.experimental.pallas import tpu_sc as plsc

M = 8
K = 1024
D = 256
SUB = D // M

TN = 256

NW = 32
CH = 128


def _csq_body(m2cb_ref, csq_ref):
    for m in range(M):
        c2 = m2cb_ref[m]
        csq_ref[m, :] = 0.25 * jnp.sum(c2 * c2, axis=1)


def _tc_body(x_ref, m2cb_ref, csq_ref, iota_ref, codes_ref):
    xs = x_ref[...]
    cross = jnp.concatenate(
        [jax.lax.dot_general(
            xs[:, m * SUB:(m + 1) * SUB], m2cb_ref[m],
            (((1,), (1,)), ((), ())), preferred_element_type=jnp.float32)
         for m in range(M)], axis=1)
    dist = cross.reshape(TN, M, K) + csq_ref[...][None]
    dmin = jnp.min(dist, axis=2, keepdims=True)
    codesf = jnp.min(
        jnp.where(dist == dmin, iota_ref[...][None], float(K)), axis=2)
    codes_ref[...] = codesf.astype(jnp.int32)


def _sc_gather(idx_hbm, table_hbm, xr_out, side_out, idx_v, rows_v, sem):
    tpw = rows_v.shape[0] // M
    wid = lax.axis_index("s") * 2 + lax.axis_index("c")
    t0 = wid * tpw
    pltpu.sync_copy(idx_hbm.at[wid], idx_v)
    copies = [
        pltpu.async_copy(table_hbm.at[idx_v.at[j]],
                         rows_v.at[pl.ds(j * CH, CH)], sem)
        for j in range(idx_v.shape[0])
    ]
    for c in copies:
        c.wait()
    for m in range(M):
        seg = rows_v.at[pl.ds(m * tpw, tpw), :]
        pltpu.sync_copy(seg, side_out.at[m, pl.ds(t0, tpw), :])
        pltpu.sync_copy(seg, xr_out.at[pl.ds(t0, tpw), pl.ds(m * SUB, SUB)])


@jax.jit
def kernel(x, codebook):
    n = x.shape[0]
    grid = n // TN
    iota2d = jax.lax.broadcasted_iota(jnp.float32, (M, K), 1)
    m2cb = -2.0 * codebook

    csq = pl.pallas_call(
        _csq_body,
        out_shape=jax.ShapeDtypeStruct((M, K), jnp.float32),
    )(m2cb)

    codes = pl.pallas_call(
        _tc_body,
        grid=(grid,),
        in_specs=[
            pl.BlockSpec((TN, D), lambda i: (i, 0)),
            pl.BlockSpec((M, K, SUB), lambda i: (0, 0, 0)),
            pl.BlockSpec((M, K), lambda i: (0, 0)),
            pl.BlockSpec((M, K), lambda i: (0, 0)),
        ],
        out_specs=pl.BlockSpec((TN, M), lambda i: (i, 0)),
        out_shape=jax.ShapeDtypeStruct((n, M), jnp.int32),
    )(x, m2cb, csq, iota2d)

    tpw = n // NW
    rpw = tpw * M
    nch = rpw // CH
    flat_mn = codes.T + jnp.arange(M, dtype=jnp.int32)[:, None] * K
    idx_w = flat_mn.reshape(M, NW, tpw).swapaxes(0, 1).reshape(NW, nch, CH)
    table = codebook.reshape(M * K, SUB)

    mesh = plsc.VectorSubcoreMesh(core_axis_name="c", subcore_axis_name="s")
    x_recon, side = pl.kernel(
        _sc_gather,
        out_type=[
            jax.ShapeDtypeStruct((n, D), jnp.float32),
            jax.ShapeDtypeStruct((M, n, SUB), jnp.float32),
        ],
        mesh=mesh,
        scratch_types=[
            pltpu.VMEM((nch, CH), jnp.int32),
            pltpu.VMEM((rpw, SUB), jnp.float32),
            pltpu.SemaphoreType.DMA,
        ],
        compiler_params=pltpu.CompilerParams(use_tc_tiling_on_sc=False),
    )(idx_w, table)

    return (x_recon, codes, side)


# speedup vs baseline: 1.0429x; 1.0140x over previous
import functools

import jax
import jax.numpy as jnp
from jax import lax
from jax.experimental import pallas as pl
from jax.experimental.pallas import tpu as pltpu
from jax.experimental.pallas import tpu_sc as plsc

M = 8
K = 1024
D = 256
SUB = D // M

TN = 256

NW = 32
CH = 128


def _csq_body(m2cb_ref, csq_ref):
    quarter = jnp.full((1, SUB), 0.25, jnp.float32)
    for m in range(M):
        c2 = m2cb_ref[m]
        csq_ref[m, :] = jax.lax.dot_general(
            quarter, c2 * c2, (((1,), (1,)), ((), ())),
            preferred_element_type=jnp.float32)[0]


def _tc_body(x_ref, m2cb_ref, csq_ref, iota_ref, codes_ref):
    xs = x_ref[...]
    cross = jnp.concatenate(
        [jax.lax.dot_general(
            xs[:, m * SUB:(m + 1) * SUB], m2cb_ref[m],
            (((1,), (1,)), ((), ())), preferred_element_type=jnp.float32)
         for m in range(M)], axis=1)
    dist = cross.reshape(TN, M, K) + csq_ref[...][None]
    dmin = jnp.min(dist, axis=2, keepdims=True)
    codesf = jnp.min(
        jnp.where(dist == dmin, iota_ref[...][None], float(K)), axis=2)
    codes_ref[...] = codesf.astype(jnp.int32)


def _sc_gather(idx_hbm, table_hbm, xr_out, side_out, idx_v, rows_v, sem):
    tpw = rows_v.shape[0] // M
    wid = lax.axis_index("s") * 2 + lax.axis_index("c")
    t0 = wid * tpw
    pltpu.sync_copy(idx_hbm.at[wid], idx_v)
    copies = [
        pltpu.async_copy(table_hbm.at[idx_v.at[j]],
                         rows_v.at[pl.ds(j * CH, CH)], sem)
        for j in range(idx_v.shape[0])
    ]
    for c in copies:
        c.wait()
    for m in range(M):
        seg = rows_v.at[pl.ds(m * tpw, tpw), :]
        pltpu.sync_copy(seg, side_out.at[m, pl.ds(t0, tpw), :])
        pltpu.sync_copy(seg, xr_out.at[pl.ds(t0, tpw), pl.ds(m * SUB, SUB)])


@jax.jit
def kernel(x, codebook):
    n = x.shape[0]
    grid = n // TN
    iota2d = jax.lax.broadcasted_iota(jnp.float32, (M, K), 1)
    m2cb = -2.0 * codebook

    csq = pl.pallas_call(
        _csq_body,
        out_shape=jax.ShapeDtypeStruct((M, K), jnp.float32),
    )(m2cb)

    codes = pl.pallas_call(
        _tc_body,
        grid=(grid,),
        in_specs=[
            pl.BlockSpec((TN, D), lambda i: (i, 0)),
            pl.BlockSpec((M, K, SUB), lambda i: (0, 0, 0)),
            pl.BlockSpec((M, K), lambda i: (0, 0)),
            pl.BlockSpec((M, K), lambda i: (0, 0)),
        ],
        out_specs=pl.BlockSpec((TN, M), lambda i: (i, 0)),
        out_shape=jax.ShapeDtypeStruct((n, M), jnp.int32),
    )(x, m2cb, csq, iota2d)

    tpw = n // NW
    rpw = tpw * M
    nch = rpw // CH
    flat_mn = codes.T + jnp.arange(M, dtype=jnp.int32)[:, None] * K
    idx_w = flat_mn.reshape(M, NW, tpw).swapaxes(0, 1).reshape(NW, nch, CH)
    table = codebook.reshape(M * K, SUB)

    mesh = plsc.VectorSubcoreMesh(core_axis_name="c", subcore_axis_name="s")
    x_recon, side = pl.kernel(
        _sc_gather,
        out_type=[
            jax.ShapeDtypeStruct((n, D), jnp.float32),
            jax.ShapeDtypeStruct((M, n, SUB), jnp.float32),
        ],
        mesh=mesh,
        scratch_types=[
            pltpu.VMEM((nch, CH), jnp.int32),
            pltpu.VMEM((rpw, SUB), jnp.float32),
            pltpu.SemaphoreType.DMA,
        ],
        compiler_params=pltpu.CompilerParams(use_tc_tiling_on_sc=False),
    )(idx_w, table)

    return (x_recon, codes, side)
